# baseline (speedup 1.0000x reference)
import jax
import jax.numpy as jnp
from jax import lax
from jax.experimental import pallas as pl
from jax.experimental.pallas import tpu as pltpu
from jax.experimental.pallas import tpu_sc as plsc

_B, _S, _F = 4, 2048, 1024
_TOT = _B * _S * _F
_NC, _NS = 2, 16
_NW = _NC * _NS
_PER_W = _TOT // _NW
_CH = 32768
_NCH = _PER_W // _CH
_VPC = _CH // 16


def _sc_rev_body(x_hbm, out_hbm, in_v, out_v):
    wid = lax.axis_index("s") * _NC + lax.axis_index("c")
    base = wid * _PER_W

    def do_chunk(ci, carry):
        off = base + ci * _CH
        pltpu.sync_copy(x_hbm.at[pl.ds(off, _CH)], in_v)

        @plsc.parallel_loop(0, _VPC, unroll=8)
        def _rev(k):
            row = k // 64
            col = (k % 64) * 16
            dst = row * _F + col
            src = row * _F + (_F - 16) - col
            out_v[pl.ds(dst, 16)] = jnp.flip(in_v[pl.ds(src, 16)], axis=0)

        pltpu.sync_copy(out_v, out_hbm.at[pl.ds(off, _CH)])
        return carry

    lax.fori_loop(0, _NCH, do_chunk, 0)


def _zeros_body(z_ref):
    z_ref[...] = jnp.zeros_like(z_ref)


def kernel(inputs):
    x = inputs.reshape(_TOT)
    mesh = plsc.VectorSubcoreMesh(
        core_axis_name="c", subcore_axis_name="s",
        num_cores=_NC, num_subcores=_NS)
    out = pl.kernel(
        _sc_rev_body,
        out_type=jax.ShapeDtypeStruct((_TOT,), jnp.float32),
        mesh=mesh,
        scratch_types=[
            pltpu.VMEM((_CH,), jnp.float32),
            pltpu.VMEM((_CH,), jnp.float32),
        ],
    )(x)
    zeros = pl.pallas_call(
        _zeros_body,
        grid=(4,),
        out_specs=pl.BlockSpec((2048, _F), lambda i: (i, 0)),
        out_shape=jax.ShapeDtypeStruct((_B * _S, _F), jnp.float32),
    )()
    return (out.reshape(_B, _S, _F), zeros.reshape(_B, _S, _F))


# speedup vs baseline: 2.0072x; 2.0072x over previous
import jax
import jax.numpy as jnp
from jax import lax
from jax.experimental import pallas as pl
from jax.experimental.pallas import tpu as pltpu
from jax.experimental.pallas import tpu_sc as plsc

_B, _S, _F = 4, 2048, 1024
_R = _B * _S
_NC, _NS = 2, 16
_NW = _NC * _NS
_RPW = _R // _NW
_CR = 32
_NCH = _RPW // _CR
_VPC = _CR * _F // 16


def _sc_rev_body(x_hbm, out_hbm, in_v, out_v):
    wid = lax.axis_index("s") * _NC + lax.axis_index("c")
    base = wid * _RPW

    def do_chunk(ci, carry):
        r0 = base + ci * _CR
        pltpu.sync_copy(x_hbm.at[pl.ds(r0, _CR)], in_v)

        @plsc.parallel_loop(0, _VPC, unroll=8)
        def _rev(k):
            row = k // 64
            col = (k % 64) * 16
            out_v[row, pl.ds(col, 16)] = jnp.flip(
                in_v[row, pl.ds((_F - 16) - col, 16)], axis=0)

        pltpu.sync_copy(out_v, out_hbm.at[pl.ds(r0, _CR)])
        return carry

    lax.fori_loop(0, _NCH, do_chunk, 0)


def _zeros_body(z_ref):
    z_ref[...] = jnp.zeros_like(z_ref)


def kernel(inputs):
    x = inputs.reshape(_R, _F)
    mesh = plsc.VectorSubcoreMesh(
        core_axis_name="c", subcore_axis_name="s",
        num_cores=_NC, num_subcores=_NS)
    out = pl.kernel(
        _sc_rev_body,
        out_type=jax.ShapeDtypeStruct((_R, _F), jnp.float32),
        mesh=mesh,
        scratch_types=[
            pltpu.VMEM((_CR, _F), jnp.float32),
            pltpu.VMEM((_CR, _F), jnp.float32),
        ],
    )(x)
    zeros = pl.pallas_call(
        _zeros_body,
        grid=(4,),
        out_specs=pl.BlockSpec((2048, _F), lambda i: (i, 0)),
        out_shape=jax.ShapeDtypeStruct((_R, _F), jnp.float32),
    )()
    return (out.reshape(_B, _S, _F), zeros.reshape(_B, _S, _F))


# speedup vs baseline: 2.3048x; 1.1483x over previous
import jax
import jax.numpy as jnp
from jax import lax
from jax.experimental import pallas as pl
from jax.experimental.pallas import tpu as pltpu
from jax.experimental.pallas import tpu_sc as plsc

_B, _S, _F = 4, 2048, 1024
_R = _B * _S
_NC, _NS = 2, 16
_NW = _NC * _NS
_RPW = _R // _NW
_CR = 16
_NCH = _RPW // _CR
_VPC = _CR * _F // 16


def _sc_rev_body(x_hbm, out_hbm, in0, in1, ot0, ot1, si0, si1, so0, so1):
    wid = lax.axis_index("s") * _NC + lax.axis_index("c")
    base = wid * _RPW
    ibufs, obufs = (in0, in1), (ot0, ot1)
    isems, osems = (si0, si1), (so0, so1)

    def src(i):
        return x_hbm.at[pl.ds(base + i * _CR, _CR)]

    def dst(i):
        return out_hbm.at[pl.ds(base + i * _CR, _CR)]

    pltpu.async_copy(src(0), ibufs[0], isems[0])
    for i in range(_NCH):
        b = i % 2
        pltpu.make_async_copy(src(i), ibufs[b], isems[b]).wait()
        if i + 1 < _NCH:
            pltpu.async_copy(src(i + 1), ibufs[1 - b], isems[1 - b])
        if i >= 2:
            pltpu.make_async_copy(obufs[b], dst(i - 2), osems[b]).wait()

        iv, ov = ibufs[b], obufs[b]

        @plsc.parallel_loop(0, _VPC, unroll=8)
        def _rev(k):
            row = k // 64
            col = (k % 64) * 16
            ov[row, pl.ds(col, 16)] = jnp.flip(
                iv[row, pl.ds((_F - 16) - col, 16)], axis=0)

        pltpu.async_copy(obufs[b], dst(i), osems[b])

    pltpu.make_async_copy(obufs[(_NCH - 2) % 2], dst(_NCH - 2),
                          osems[(_NCH - 2) % 2]).wait()
    pltpu.make_async_copy(obufs[(_NCH - 1) % 2], dst(_NCH - 1),
                          osems[(_NCH - 1) % 2]).wait()


def _zeros_body(z_ref):
    z_ref[...] = jnp.zeros_like(z_ref)


def kernel(inputs):
    x = inputs.reshape(_R, _F)
    mesh = plsc.VectorSubcoreMesh(
        core_axis_name="c", subcore_axis_name="s",
        num_cores=_NC, num_subcores=_NS)
    out = pl.kernel(
        _sc_rev_body,
        out_type=jax.ShapeDtypeStruct((_R, _F), jnp.float32),
        mesh=mesh,
        scratch_types=[
            pltpu.VMEM((_CR, _F), jnp.float32),
            pltpu.VMEM((_CR, _F), jnp.float32),
            pltpu.VMEM((_CR, _F), jnp.float32),
            pltpu.VMEM((_CR, _F), jnp.float32),
            pltpu.SemaphoreType.DMA,
            pltpu.SemaphoreType.DMA,
            pltpu.SemaphoreType.DMA,
            pltpu.SemaphoreType.DMA,
        ],
    )(x)
    zeros = pl.pallas_call(
        _zeros_body,
        grid=(4,),
        out_specs=pl.BlockSpec((2048, _F), lambda i: (i, 0)),
        out_shape=jax.ShapeDtypeStruct((_R, _F), jnp.float32),
    )()
    return (out.reshape(_B, _S, _F), zeros.reshape(_B, _S, _F))


# speedup vs baseline: 2.4754x; 1.0740x over previous
import jax
import jax.numpy as jnp
from jax import lax
from jax.experimental import pallas as pl
from jax.experimental.pallas import tpu as pltpu
from jax.experimental.pallas import tpu_sc as plsc

_B, _S, _F = 4, 2048, 1024
_R = _B * _S
_NC, _NS = 2, 16
_NW = _NC * _NS
_RPW = _R // _NW
_CR = 16
_NCH = _RPW // _CR
_VPC = _CR * _F // 16


def _sc_rev_body(x_hbm, out_hbm, in0, in1, in2, ot0, ot1, ot2,
                 si0, si1, si2, so0, so1, so2):
    wid = lax.axis_index("s") * _NC + lax.axis_index("c")
    base = wid * _RPW
    ibufs, obufs = (in0, in1, in2), (ot0, ot1, ot2)
    isems, osems = (si0, si1, si2), (so0, so1, so2)

    def src(i):
        return x_hbm.at[pl.ds(base + i * _CR, _CR)]

    def dst(i):
        return out_hbm.at[pl.ds(base + i * _CR, _CR)]

    pltpu.async_copy(src(0), ibufs[0], isems[0])
    pltpu.async_copy(src(1), ibufs[1], isems[1])
    for i in range(_NCH):
        b = i % 3
        pltpu.make_async_copy(src(i), ibufs[b], isems[b]).wait()
        if i + 2 < _NCH:
            b2 = (i + 2) % 3
            pltpu.async_copy(src(i + 2), ibufs[b2], isems[b2])
        if i >= 3:
            pltpu.make_async_copy(obufs[b], dst(i - 3), osems[b]).wait()

        iv, ov = ibufs[b], obufs[b]

        @plsc.parallel_loop(0, _VPC, unroll=8)
        def _rev(k):
            row = k // 64
            col = (k % 64) * 16
            ov[row, pl.ds(col, 16)] = jnp.flip(
                iv[row, pl.ds((_F - 16) - col, 16)], axis=0)

        pltpu.async_copy(obufs[b], dst(i), osems[b])

    for i in range(_NCH - 3, _NCH):
        pltpu.make_async_copy(obufs[i % 3], dst(i), osems[i % 3]).wait()


def _zeros_body(z_ref):
    z_ref[...] = jnp.zeros_like(z_ref)


def kernel(inputs):
    x = inputs.reshape(_R, _F)
    mesh = plsc.VectorSubcoreMesh(
        core_axis_name="c", subcore_axis_name="s",
        num_cores=_NC, num_subcores=_NS)
    out = pl.kernel(
        _sc_rev_body,
        out_type=jax.ShapeDtypeStruct((_R, _F), jnp.float32),
        mesh=mesh,
        scratch_types=(
            [pltpu.VMEM((_CR, _F), jnp.float32)] * 6
            + [pltpu.SemaphoreType.DMA] * 6),
    )(x)
    zeros = pl.pallas_call(
        _zeros_body,
        grid=(4,),
        out_specs=pl.BlockSpec((2048, _F), lambda i: (i, 0)),
        out_shape=jax.ShapeDtypeStruct((_R, _F), jnp.float32),
    )()
    return (out.reshape(_B, _S, _F), zeros.reshape(_B, _S, _F))
